# baseline (speedup 1.0000x reference)
import functools

import numpy as np
import jax
import jax.numpy as jnp
from jax import lax
from jax.experimental import pallas as pl
from jax.experimental.pallas import tpu as pltpu
from jax.experimental.pallas import tpu_sc as plsc

B, C, X = 4, 2048, 1024
BC = B * C
H, DH = 16, 64
KC = 204

_PERM = np.zeros(3 * X, np.int64)
for _i in range(H):
    _PERM[_i * 128:_i * 128 + 64] = np.arange(_i * 64, (_i + 1) * 64)
    _PERM[_i * 128 + 64:(_i + 1) * 128] = np.arange(X + _i * 64, X + (_i + 1) * 64)
_PERM[2 * X:] = np.arange(2 * X, 3 * X)

MB = 512
GPB = MB * H // 128

_NEG = -3.4e38


def _fused_kernel(x_ref, w_ref, b_ref, sa_ref, v_ref):
    qkv = jnp.dot(x_ref[...], w_ref[...],
                  preferred_element_type=jnp.float32) + b_ref[...]
    v_ref[...] = qkv[:, 2 * X:]
    qk = qkv[:, :2 * X].reshape(MB, H, 128).reshape(MB * H, 128)

    lane = lax.broadcasted_iota(jnp.int32, (128, 128), 1)
    br = lax.broadcasted_iota(jnp.int32, (128, 128), 0) // 16
    mask = br == (lane // 16)
    colj = lane % 16
    j16 = lax.broadcasted_iota(jnp.int32, (128, 16), 1)
    for g in range(GPB):
        rows = qk[g * 128:(g + 1) * 128]
        qg = jnp.where(lane < 64, rows, 0.0)
        kg = pltpu.roll(jnp.where(lane >= 64, rows, 0.0), 64, 1)
        a = lax.dot_general(qg, kg, (((1,), (1,)), ((), ())),
                            preferred_element_type=jnp.float32)
        am = jnp.where(mask, a, _NEG)
        m = jnp.max(am, axis=1, keepdims=True)
        jl = jnp.min(jnp.where(am == m, colj, 1 << 30), axis=1, keepdims=True)
        sa_ref[pl.ds(g * 128, 128), :] = (j16 == jl).astype(jnp.float32)


def _fused(x2, Wp, bp):
    return pl.pallas_call(
        _fused_kernel,
        grid=(BC // MB,),
        in_specs=[
            pl.BlockSpec((MB, X), lambda i: (i, 0)),
            pl.BlockSpec((X, 3 * X), lambda i: (0, 0)),
            pl.BlockSpec((1, 3 * X), lambda i: (0, 0)),
        ],
        out_specs=[
            pl.BlockSpec((MB * H, H), lambda i: (i, 0)),
            pl.BlockSpec((MB, X), lambda i: (i, 0)),
        ],
        out_shape=[
            jax.ShapeDtypeStruct((BC * H, H), jnp.float32),
            jax.ShapeDtypeStruct((BC, X), jnp.float32),
        ],
    )(x2, Wp, bp)


def _topk_kernel(ci_ref, chi_ref):
    vals0 = ci_ref[...]
    lane = lax.broadcasted_iota(jnp.int32, (8, C), 1)
    lane256 = lax.broadcasted_iota(jnp.int32, (8, 256), 1)

    def body(r, carry):
        vals, acc = carry
        m = jnp.max(vals, axis=1, keepdims=True)
        idx = jnp.min(jnp.where(vals == m, lane, 1 << 30),
                      axis=1, keepdims=True)
        acc = jnp.where(lane256 == r, idx, acc)
        vals = jnp.where(lane == idx, _NEG, vals)
        return vals, acc

    _, acc = lax.fori_loop(0, KC, body,
                           (vals0, jnp.zeros((8, 256), jnp.int32)))
    chi_ref[...] = acc


def _topk(ci_pad):
    return pl.pallas_call(
        _topk_kernel,
        grid=(1,),
        in_specs=[pl.BlockSpec((8, C), lambda i: (0, 0))],
        out_specs=pl.BlockSpec((8, 256), lambda i: (0, 0)),
        out_shape=jax.ShapeDtypeStruct((8, 256), jnp.int32),
    )(ci_pad)


_info = plsc.get_sparse_core_info()
_NW = _info.num_cores * _info.num_subcores
GN = 1024
BPW = GN // _NW


def _make_gather():
    mesh = plsc.VectorSubcoreMesh(core_axis_name="c", subcore_axis_name="s")

    @functools.partial(
        pl.kernel, mesh=mesh,
        out_type=jax.ShapeDtypeStruct((GN, 1024), jnp.float32),
        compiler_params=pltpu.CompilerParams(use_tc_tiling_on_sc=True),
        scratch_types=[
            pltpu.VMEM((BPW,), jnp.int32),
            pltpu.VMEM((BPW, 1024), jnp.float32),
            pltpu.SemaphoreType.DMA,
        ],
    )
    def k(table_hbm, idx_hbm, out_hbm, idx_v, rows_v, sem):
        wid = lax.axis_index("s") * _info.num_cores + lax.axis_index("c")
        base = wid * BPW
        pltpu.sync_copy(idx_hbm.at[pl.ds(base, BPW)], idx_v)
        pltpu.async_copy(table_hbm.at[idx_v], rows_v, sem).wait()
        pltpu.sync_copy(rows_v, out_hbm.at[pl.ds(base, BPW)])

    return k


_gather = _make_gather()


def kernel(x, W, b):
    Wqkp = W[:, :2 * X].reshape(X, 2, H, DH).transpose(0, 2, 1, 3).reshape(X, 2 * X)
    Wp = jnp.concatenate([Wqkp, W[:, 2 * X:]], axis=1)
    bqkp = b[:2 * X].reshape(2, H, DH).transpose(1, 0, 2).reshape(2 * X)
    bp = jnp.concatenate([bqkp, b[2 * X:]]).reshape(1, 3 * X)
    sa, v = _fused(x.reshape(BC, X), Wp, bp)
    sa3 = sa.reshape(BC, H, H)
    v3 = v.reshape(BC, H, DH)

    out = jnp.matmul(sa3, v3)
    out_t = jnp.transpose(out, (0, 2, 1)).reshape(B, C, X)
    ci = jnp.linalg.norm(out_t, axis=-1)

    ci_pad = jnp.concatenate(
        [ci, jnp.full((4, C), -1.0, jnp.float32)], axis=0)
    chi = _topk(ci_pad)[:B, :KC]

    tix = (chi + (jnp.arange(B, dtype=jnp.int32) * C)[:, None]).reshape(B * KC)
    tix = jnp.concatenate(
        [tix, jnp.zeros((GN - B * KC,), jnp.int32)], axis=0)
    gathered = _gather(out_t.reshape(BC, X), tix)
    sparse_feat = gathered[:B * KC].reshape(B, KC, X)
    return sparse_feat, chi, KC


# speedup vs baseline: 1.0540x; 1.0540x over previous
import functools

import jax
import jax.numpy as jnp
from jax import lax
from jax.experimental import pallas as pl
from jax.experimental.pallas import tpu as pltpu
from jax.experimental.pallas import tpu_sc as plsc

B, C, X = 4, 2048, 1024
BC = B * C
H, DH = 16, 64
KC = 204

MB = 256
GPB = MB * H // 128

_NEG = -3.4e38


def _fused_kernel(x_ref, w_ref, b_ref, sa_ref, v_ref):
    qkv = jnp.dot(x_ref[...], w_ref[...],
                  preferred_element_type=jnp.float32) + b_ref[...]
    v_ref[...] = qkv[:, 2 * X:]
    qk = qkv[:, :2 * X].reshape(MB, H, 128).reshape(MB * H, 128)

    lane = lax.broadcasted_iota(jnp.int32, (128, 128), 1)
    br = lax.broadcasted_iota(jnp.int32, (128, 128), 0) // 16
    mask = br == (lane // 16)
    colj = lane % 16
    j16 = lax.broadcasted_iota(jnp.int32, (128, 16), 1)
    for g in range(GPB):
        rows = qk[g * 128:(g + 1) * 128]
        qg = jnp.where(lane < 64, rows, 0.0)
        kg = pltpu.roll(jnp.where(lane >= 64, rows, 0.0), 64, 1)
        a = lax.dot_general(qg, kg, (((1,), (1,)), ((), ())),
                            preferred_element_type=jnp.float32)
        am = jnp.where(mask, a, _NEG)
        m = jnp.max(am, axis=1, keepdims=True)
        jl = jnp.min(jnp.where(am == m, colj, 1 << 30), axis=1, keepdims=True)
        sa_ref[pl.ds(g * 128, 128), :] = (j16 == jl).astype(jnp.float32)


def _fused(x2, Wp, bp):
    return pl.pallas_call(
        _fused_kernel,
        grid=(BC // MB,),
        in_specs=[
            pl.BlockSpec((MB, X), lambda i: (i, 0)),
            pl.BlockSpec((X, 3 * X), lambda i: (0, 0)),
            pl.BlockSpec((1, 3 * X), lambda i: (0, 0)),
        ],
        out_specs=[
            pl.BlockSpec((MB * H, H), lambda i: (i, 0)),
            pl.BlockSpec((MB, X), lambda i: (i, 0)),
        ],
        out_shape=[
            jax.ShapeDtypeStruct((BC * H, H), jnp.float32),
            jax.ShapeDtypeStruct((BC, X), jnp.float32),
        ],
    )(x2, Wp, bp)


def _topk_kernel(ci_ref, chi_ref):
    vals0 = ci_ref[...]
    lane = lax.broadcasted_iota(jnp.int32, (8, C), 1)
    lane256 = lax.broadcasted_iota(jnp.int32, (8, 256), 1)

    def body(r, carry):
        vals, acc = carry
        m = jnp.max(vals, axis=1, keepdims=True)
        idx = jnp.min(jnp.where(vals == m, lane, 1 << 30),
                      axis=1, keepdims=True)
        acc = jnp.where(lane256 == r, idx, acc)
        vals = jnp.where(lane == idx, _NEG, vals)
        return vals, acc

    _, acc = lax.fori_loop(0, KC, body,
                           (vals0, jnp.zeros((8, 256), jnp.int32)))
    chi_ref[...] = acc


def _topk(ci_pad):
    return pl.pallas_call(
        _topk_kernel,
        grid=(1,),
        in_specs=[pl.BlockSpec((8, C), lambda i: (0, 0))],
        out_specs=pl.BlockSpec((8, 256), lambda i: (0, 0)),
        out_shape=jax.ShapeDtypeStruct((8, 256), jnp.int32),
    )(ci_pad)


_NC, _NS = 2, 16
_NW = _NC * _NS
GN = 1024
BPW = GN // _NW


def _make_gather():
    mesh = plsc.VectorSubcoreMesh(core_axis_name="c", subcore_axis_name="s")

    @functools.partial(
        pl.kernel, mesh=mesh,
        out_type=jax.ShapeDtypeStruct((GN, 1024), jnp.float32),
        scratch_types=[
            pltpu.VMEM((BPW,), jnp.int32),
            pltpu.VMEM((BPW, 1024), jnp.float32),
            pltpu.SemaphoreType.DMA,
        ],
    )
    def k(table_hbm, idx_hbm, out_hbm, idx_v, rows_v, sem):
        wid = lax.axis_index("s") * _NC + lax.axis_index("c")
        base = wid * BPW
        pltpu.sync_copy(idx_hbm.at[pl.ds(base, BPW)], idx_v)
        pltpu.async_copy(table_hbm.at[idx_v], rows_v, sem).wait()
        pltpu.sync_copy(rows_v, out_hbm.at[pl.ds(base, BPW)])

    return k


def kernel(x, W, b):
    Wqkp = W[:, :2 * X].reshape(X, 2, H, DH).transpose(0, 2, 1, 3).reshape(X, 2 * X)
    Wp = jnp.concatenate([Wqkp, W[:, 2 * X:]], axis=1)
    bqkp = b[:2 * X].reshape(2, H, DH).transpose(1, 0, 2).reshape(2 * X)
    bp = jnp.concatenate([bqkp, b[2 * X:]]).reshape(1, 3 * X)
    sa, v = _fused(x.reshape(BC, X), Wp, bp)
    sa3 = sa.reshape(BC, H, H)
    v3 = v.reshape(BC, H, DH)

    out = jnp.matmul(sa3, v3)
    out_t = jnp.transpose(out, (0, 2, 1)).reshape(B, C, X)
    ci = jnp.linalg.norm(out_t, axis=-1)

    ci_pad = jnp.concatenate(
        [ci, jnp.full((4, C), -1.0, jnp.float32)], axis=0)
    chi = _topk(ci_pad)[:B, :KC]

    tix = (chi + (jnp.arange(B, dtype=jnp.int32) * C)[:, None]).reshape(B * KC)
    tix = jnp.concatenate(
        [tix, jnp.zeros((GN - B * KC,), jnp.int32)], axis=0)
    gathered = _make_gather()(out_t.reshape(BC, X), tix)
    sparse_feat = gathered[:B * KC].reshape(B, KC, X)
    return sparse_feat, chi, KC
